# baseline (speedup 1.0000x reference)
import functools

import jax
import jax.numpy as jnp
from jax import lax
from jax.experimental import pallas as pl
from jax.experimental.pallas import tpu as pltpu
from jax.experimental.pallas import tpu_sc as plsc

_NC = 2
_NS = 16


def _make_segsum(d, k, steps0, steps1, n_acc):
    rpt = n_acc // _NS
    mesh = plsc.VectorSubcoreMesh(core_axis_name="c", subcore_axis_name="s")

    @functools.partial(
        pl.kernel,
        out_type=jax.ShapeDtypeStruct((_NC, n_acc, d), jnp.float32),
        mesh=mesh,
        compiler_params=pltpu.CompilerParams(use_tc_tiling_on_sc=False),
        scratch_types=[
            pltpu.VMEM((k,), jnp.int32),
            pltpu.VMEM((k,), jnp.int32),
            pltpu.VMEM((k, d), jnp.float32),
            pltpu.VMEM_SHARED((n_acc, d), jnp.float32),
            pltpu.SemaphoreType.DMA,
        ],
    )
    def seg(u_hbm, src_hbm, dst_hbm, zero_hbm, out_hbm,
            src_v, dst_v, rows_v, acc_sh, sem):
        c = lax.axis_index("c")
        s = lax.axis_index("s")
        base = jnp.where(c == 0, s * steps0, _NS * steps0 + s * steps1)
        n_my = jnp.where(c == 0, steps0, steps1)
        r0 = s * rpt
        pltpu.sync_copy(zero_hbm, acc_sh.at[pl.ds(r0, rpt)])
        plsc.subcore_barrier()

        @pl.loop(0, n_my)
        def _(t):
            pltpu.sync_copy(src_hbm.at[base + t], src_v)
            pltpu.sync_copy(dst_hbm.at[base + t], dst_v)
            pltpu.async_copy(u_hbm.at[src_v], rows_v, sem).wait()
            pltpu.sync_copy(rows_v, acc_sh.at[dst_v], add=True)

        plsc.subcore_barrier()
        pltpu.sync_copy(acc_sh.at[pl.ds(r0, rpt)],
                        out_hbm.at[c, pl.ds(r0, rpt)])

    return seg


def _lrelu(v):
    return jnp.where(v >= 0, v, 0.01 * v)


def _mm(a, b_t):
    return lax.dot_general(a, b_t, (((1,), (1,)), ((), ())),
                           preferred_element_type=jnp.float32)


def _layer_body(p_ref, z_ref, b_ref, wr_ref, wt_ref, zn_ref):
    zn_ref[...] = _lrelu(_mm(p_ref[0] + p_ref[1], wr_ref[...]) + b_ref[...]
                         + _mm(z_ref[...], wt_ref[...]))


def _layer_skip_body(p_ref, z_ref, b_ref, wr_ref, wt_ref, wrn_ref,
                     zs_ref, un_ref):
    zn = _lrelu(_mm(p_ref[0] + p_ref[1], wr_ref[...]) + b_ref[...]
                + _mm(z_ref[...], wt_ref[...]))
    zs = zn + z_ref[...]
    zs_ref[...] = zs
    un_ref[...] = _mm(zs, wrn_ref[...])


def _final_body(p_ref, z_ref, b_ref, wt_ref, o_ref):
    o_ref[...] = _lrelu(p_ref[0] + p_ref[1] + b_ref[...]
                        + _mm(z_ref[...], wt_ref[...]))


def _blk(shape, imap):
    return pl.BlockSpec(shape, imap)


_ROWS = 1000


def _run_layer0(parts, z, b, wr, wt):
    n, d = z.shape
    return pl.pallas_call(
        _layer_body,
        grid=(n // _ROWS,),
        in_specs=[_blk((2, _ROWS, d), lambda i: (0, i, 0)),
                  _blk((_ROWS, d), lambda i: (i, 0)),
                  _blk((1, d), lambda i: (0, 0)),
                  _blk((d, d), lambda i: (0, 0)),
                  _blk((d, d), lambda i: (0, 0))],
        out_specs=_blk((_ROWS, d), lambda i: (i, 0)),
        out_shape=jax.ShapeDtypeStruct((n, d), jnp.float32),
    )(parts, z, b.reshape(1, -1), wr, wt)


def _run_layer1(parts, z, b, wr, wt, wrn, dnext):
    n, d = z.shape
    return pl.pallas_call(
        _layer_skip_body,
        grid=(n // _ROWS,),
        in_specs=[_blk((2, _ROWS, d), lambda i: (0, i, 0)),
                  _blk((_ROWS, d), lambda i: (i, 0)),
                  _blk((1, d), lambda i: (0, 0)),
                  _blk((d, d), lambda i: (0, 0)),
                  _blk((d, d), lambda i: (0, 0)),
                  _blk((dnext, d), lambda i: (0, 0))],
        out_specs=[_blk((_ROWS, d), lambda i: (i, 0)),
                   _blk((_ROWS, dnext), lambda i: (i, 0))],
        out_shape=[jax.ShapeDtypeStruct((n, d), jnp.float32),
                   jax.ShapeDtypeStruct((n, dnext), jnp.float32)],
    )(parts, z, b.reshape(1, -1), wr, wt, wrn)


def _run_final(parts, z, b, wt):
    n, d = z.shape
    dp = parts.shape[2]
    return pl.pallas_call(
        _final_body,
        grid=(n // _ROWS,),
        in_specs=[_blk((2, _ROWS, dp), lambda i: (0, i, 0)),
                  _blk((_ROWS, d), lambda i: (i, 0)),
                  _blk((1, dp), lambda i: (0, 0)),
                  _blk((dp, d), lambda i: (0, 0))],
        out_specs=_blk((_ROWS, dp), lambda i: (i, 0)),
        out_shape=jax.ShapeDtypeStruct((n, dp), jnp.float32),
    )(parts, z, b.reshape(1, -1), wt)


@jax.jit
def kernel(x, edge_index, W_rel_0, b_rel_0, W_root_0, W_rel_1, b_rel_1,
           W_root_1, W_rel_2, b_rel_2, W_root_2):
    n = x.shape[0]
    e = edge_index.shape[1]
    n_acc = -(-(n + 1) // (_NS * 8)) * (_NS * 8)

    k128, k16 = 128, 512
    quant = 2 * _NS * k16
    e_pad = -(-e // quant) * quant
    npad = e_pad - e

    src_flat = jnp.concatenate([edge_index[0],
                                jnp.zeros((npad,), jnp.int32)])
    dst_flat = jnp.concatenate([edge_index[1],
                                jnp.full((npad,), n_acc - 1, jnp.int32)])

    half = e_pad // (2 * _NS * k128)
    steps1 = max(1, int(2 * half * 0.386))
    steps0 = 2 * half - steps1
    srcw = src_flat.reshape(-1, k128)
    dstw = dst_flat.reshape(-1, k128)

    half16 = e_pad // (2 * _NS * k16)
    srcn = src_flat.reshape(-1, k16)
    dstn = dst_flat.reshape(-1, k16)

    zero128 = jnp.zeros((n_acc // _NS, 128), jnp.float32)
    seg128 = _make_segsum(128, k128, steps0, steps1, n_acc)

    d2 = 16
    wr2 = jnp.zeros((d2, 128), jnp.float32).at[:4].set(W_rel_2)
    wt2 = jnp.zeros((d2, 128), jnp.float32).at[:4].set(W_root_2)
    b2 = jnp.zeros((d2,), jnp.float32).at[:4].set(b_rel_2)

    p0 = seg128(x, srcw, dstw, zero128)[:, :n]
    z0 = _run_layer0(p0, x, b_rel_0, W_rel_0, W_root_0)

    p1 = seg128(z0, srcw, dstw, zero128)[:, :n]
    zs, u2 = _run_layer1(p1, z0, b_rel_1, W_rel_1, W_root_1, wr2, d2)

    zero16 = jnp.zeros((n_acc // _NS, d2), jnp.float32)
    p2 = _make_segsum(d2, k16, half16, half16, n_acc)(
        u2, srcn, dstn, zero16)[:, :n]
    out = _run_final(p2, zs, b2, wt2)
    return out[:, :4]


# speedup vs baseline: 1.3098x; 1.3098x over previous
import functools

import jax
import jax.numpy as jnp
from jax import lax
from jax.experimental import pallas as pl
from jax.experimental.pallas import tpu as pltpu
from jax.experimental.pallas import tpu_sc as plsc

_NC = 2
_NS = 16


def _make_segsum(d, k, steps0, steps1, n_acc):
    rpt = n_acc // _NS
    mesh = plsc.VectorSubcoreMesh(core_axis_name="c", subcore_axis_name="s")

    @functools.partial(
        pl.kernel,
        out_type=jax.ShapeDtypeStruct((_NC, n_acc, d), jnp.float32),
        mesh=mesh,
        compiler_params=pltpu.CompilerParams(use_tc_tiling_on_sc=False),
        scratch_types=[
            pltpu.VMEM((k,), jnp.int32),
            pltpu.VMEM((k,), jnp.int32),
            pltpu.VMEM((k, d), jnp.float32),
            pltpu.VMEM_SHARED((n_acc, d), jnp.float32),
            pltpu.SemaphoreType.DMA,
        ],
    )
    def seg(u_hbm, src_hbm, dst_hbm, zero_hbm, out_hbm,
            src_v, dst_v, rows_v, acc_sh, sem):
        c = lax.axis_index("c")
        s = lax.axis_index("s")
        base = jnp.where(c == 0, s * steps0, _NS * steps0 + s * steps1)
        n_my = jnp.where(c == 0, steps0, steps1)
        r0 = s * rpt
        pltpu.sync_copy(zero_hbm, acc_sh.at[pl.ds(r0, rpt)])
        plsc.subcore_barrier()

        @pl.loop(0, n_my)
        def _(t):
            pltpu.sync_copy(src_hbm.at[base + t], src_v)
            pltpu.sync_copy(dst_hbm.at[base + t], dst_v)
            pltpu.async_copy(u_hbm.at[src_v], rows_v, sem).wait()
            pltpu.sync_copy(rows_v, acc_sh.at[dst_v], add=True)

        plsc.subcore_barrier()
        pltpu.sync_copy(acc_sh.at[pl.ds(r0, rpt)],
                        out_hbm.at[c, pl.ds(r0, rpt)])

    return seg


def _lrelu(v):
    return jnp.where(v >= 0, v, 0.01 * v)


def _mm(a, b_t):
    return lax.dot_general(a, b_t, (((1,), (1,)), ((), ())),
                           preferred_element_type=jnp.float32)


def _layer_body(p_ref, z_ref, b_ref, wr_ref, wt_ref, zn_ref):
    zn_ref[...] = _lrelu(_mm(p_ref[0] + p_ref[1], wr_ref[...]) + b_ref[...]
                         + _mm(z_ref[...], wt_ref[...]))


def _layer_skip_body(p_ref, z_ref, b_ref, wr_ref, wt_ref, wrn_ref,
                     zs_ref, un_ref):
    zn = _lrelu(_mm(p_ref[0] + p_ref[1], wr_ref[...]) + b_ref[...]
                + _mm(z_ref[...], wt_ref[...]))
    zs = zn + z_ref[...]
    zs_ref[...] = zs
    un_ref[...] = _mm(zs, wrn_ref[...])


def _final_body(p_ref, z_ref, b_ref, wt_ref, o_ref):
    o_ref[...] = _lrelu(p_ref[0] + p_ref[1] + b_ref[...]
                        + _mm(z_ref[...], wt_ref[...]))


def _blk(shape, imap):
    return pl.BlockSpec(shape, imap)


_ROWS = 1000


def _run_layer0(parts, z, b, wr, wt):
    n, d = z.shape
    return pl.pallas_call(
        _layer_body,
        grid=(n // _ROWS,),
        in_specs=[_blk((2, _ROWS, d), lambda i: (0, i, 0)),
                  _blk((_ROWS, d), lambda i: (i, 0)),
                  _blk((1, d), lambda i: (0, 0)),
                  _blk((d, d), lambda i: (0, 0)),
                  _blk((d, d), lambda i: (0, 0))],
        out_specs=_blk((_ROWS, d), lambda i: (i, 0)),
        out_shape=jax.ShapeDtypeStruct((n, d), jnp.float32),
    )(parts, z, b.reshape(1, -1), wr, wt)


def _run_layer1(parts, z, b, wr, wt, wrn, dnext):
    n, d = z.shape
    return pl.pallas_call(
        _layer_skip_body,
        grid=(n // _ROWS,),
        in_specs=[_blk((2, _ROWS, d), lambda i: (0, i, 0)),
                  _blk((_ROWS, d), lambda i: (i, 0)),
                  _blk((1, d), lambda i: (0, 0)),
                  _blk((d, d), lambda i: (0, 0)),
                  _blk((d, d), lambda i: (0, 0)),
                  _blk((dnext, d), lambda i: (0, 0))],
        out_specs=[_blk((_ROWS, d), lambda i: (i, 0)),
                   _blk((_ROWS, dnext), lambda i: (i, 0))],
        out_shape=[jax.ShapeDtypeStruct((n, d), jnp.float32),
                   jax.ShapeDtypeStruct((n, dnext), jnp.float32)],
    )(parts, z, b.reshape(1, -1), wr, wt, wrn)


def _run_final(parts, z, b, wt):
    n, d = z.shape
    dp = parts.shape[2]
    return pl.pallas_call(
        _final_body,
        grid=(n // _ROWS,),
        in_specs=[_blk((2, _ROWS, dp), lambda i: (0, i, 0)),
                  _blk((_ROWS, d), lambda i: (i, 0)),
                  _blk((1, dp), lambda i: (0, 0)),
                  _blk((dp, d), lambda i: (0, 0))],
        out_specs=_blk((_ROWS, dp), lambda i: (i, 0)),
        out_shape=jax.ShapeDtypeStruct((n, dp), jnp.float32),
    )(parts, z, b.reshape(1, -1), wt)


@jax.jit
def kernel(x, edge_index, W_rel_0, b_rel_0, W_root_0, W_rel_1, b_rel_1,
           W_root_1, W_rel_2, b_rel_2, W_root_2):
    n = x.shape[0]
    e = edge_index.shape[1]
    n_acc = -(-(n + 1) // (_NS * 8)) * (_NS * 8)

    k128, k16 = 128, 128
    quant = 2 * _NS * k16
    e_pad = -(-e // quant) * quant
    npad = e_pad - e

    src_flat = jnp.concatenate([edge_index[0],
                                jnp.zeros((npad,), jnp.int32)])
    dst_flat = jnp.concatenate([edge_index[1],
                                jnp.full((npad,), n_acc - 1, jnp.int32)])

    half = e_pad // (2 * _NS * k128)
    steps1 = max(1, int(2 * half * 0.386))
    steps0 = 2 * half - steps1
    srcw = src_flat.reshape(-1, k128)
    dstw = dst_flat.reshape(-1, k128)

    half16 = e_pad // (2 * _NS * k16)
    srcn = src_flat.reshape(-1, k16)
    dstn = dst_flat.reshape(-1, k16)

    zero128 = jnp.zeros((n_acc // _NS, 128), jnp.float32)
    seg128 = _make_segsum(128, k128, steps0, steps1, n_acc)

    d2 = 16
    wr2 = jnp.zeros((d2, 128), jnp.float32).at[:4].set(W_rel_2)
    wt2 = jnp.zeros((d2, 128), jnp.float32).at[:4].set(W_root_2)
    b2 = jnp.zeros((d2,), jnp.float32).at[:4].set(b_rel_2)

    p0 = seg128(x, srcw, dstw, zero128)[:, :n]
    z0 = _run_layer0(p0, x, b_rel_0, W_rel_0, W_root_0)

    p1 = seg128(z0, srcw, dstw, zero128)[:, :n]
    zs, u2 = _run_layer1(p1, z0, b_rel_1, W_rel_1, W_root_1, wr2, d2)

    zero16 = jnp.zeros((n_acc // _NS, d2), jnp.float32)
    p2 = _make_segsum(d2, k16, half16, half16, n_acc)(
        u2, srcn, dstn, zero16)[:, :n]
    out = _run_final(p2, zs, b2, wt2)
    return out[:, :4]
